# baseline (speedup 1.0000x reference)
import functools

import jax
import jax.numpy as jnp
import numpy as np
from jax.experimental import pallas as pl
from jax.experimental.pallas import tpu as pltpu

N = 10000
BR = 400
BRC = 400


def _dinv(deg):
    return jnp.where(deg > 0, jax.lax.rsqrt(deg), 0.0)


def _cast_deg_body(a_ref, a16_ref, di_ref):
    blk = a_ref[...]
    a16_ref[...] = blk.astype(jnp.bfloat16)
    di_ref[...] = _dinv(jnp.sum(blk, axis=1, keepdims=True))


def _cast_deg(a):
    return pl.pallas_call(
        _cast_deg_body,
        grid=(N // BRC,),
        in_specs=[pl.BlockSpec((BRC, N), lambda i: (i, 0))],
        out_specs=[pl.BlockSpec((BRC, N), lambda i: (i, 0)),
                   pl.BlockSpec((BRC, 1), lambda i: (i, 0))],
        out_shape=[jax.ShapeDtypeStruct((N, N), jnp.bfloat16),
                   jax.ShapeDtypeStruct((N, 1), jnp.float32)],
    )(a)


def _deg_body(a_ref, m_ref, o_ref):
    deg = jnp.dot(a_ref[...], m_ref[...], preferred_element_type=jnp.float32)
    o_ref[...] = _dinv(deg)


def _deg_pass(a16, m):
    return pl.pallas_call(
        _deg_body,
        grid=(N // BR,),
        in_specs=[
            pl.BlockSpec((BR, N), lambda i: (i, 0)),
            pl.BlockSpec((N, 1), lambda i: (0, 0)),
        ],
        out_specs=pl.BlockSpec((BR, 1), lambda i: (i, 0)),
        out_shape=jax.ShapeDtypeStruct((N, 1), jnp.float32),
    )(a16, m.astype(a16.dtype))


def _conv1_body(a_ref, dif_ref, dib_ref, xw_ref, xb_ref, p_ref,
                o_feat, o_y, o_cs, v_scr, vr_scr):
    @pl.when(pl.program_id(0) == 0)
    def _():
        vf = dif_ref[...] * xw_ref[...]
        v16 = vf.astype(jnp.bfloat16)
        v_scr[...] = v16
        vr_scr[...] = (vf - v16.astype(jnp.float32)).astype(jnp.bfloat16)

    z = (jnp.dot(a_ref[...], v_scr[...], preferred_element_type=jnp.float32)
         + jnp.dot(a_ref[...], vr_scr[...],
                   preferred_element_type=jnp.float32))
    feat = jax.nn.relu(dib_ref[...] * z + xb_ref[...])
    o_feat[...] = feat
    o_y[...] = jnp.dot(feat, p_ref[...], preferred_element_type=jnp.float32)
    o_cs[...] = jnp.sum(feat, axis=0, keepdims=True)[None]


def _conv1(a16, di, xw, xb, p_col):
    h = xw.shape[1]
    return pl.pallas_call(
        _conv1_body,
        grid=(N // BR,),
        in_specs=[
            pl.BlockSpec((BR, N), lambda i: (i, 0)),
            pl.BlockSpec((N, 1), lambda i: (0, 0)),
            pl.BlockSpec((BR, 1), lambda i: (i, 0)),
            pl.BlockSpec((N, h), lambda i: (0, 0)),
            pl.BlockSpec((BR, h), lambda i: (i, 0)),
            pl.BlockSpec((h, 1), lambda i: (0, 0)),
        ],
        out_specs=[
            pl.BlockSpec((BR, h), lambda i: (i, 0)),
            pl.BlockSpec((BR, 1), lambda i: (i, 0)),
            pl.BlockSpec((1, 1, h), lambda i: (i, 0, 0)),
        ],
        out_shape=[
            jax.ShapeDtypeStruct((N, h), jnp.float32),
            jax.ShapeDtypeStruct((N, 1), jnp.float32),
            jax.ShapeDtypeStruct((N // BR, 1, h), jnp.float32),
        ],
        scratch_shapes=[pltpu.VMEM((N, h), jnp.bfloat16),
                        pltpu.VMEM((N, h), jnp.bfloat16)],
    )(a16, di, di, xw, xb, p_col)


def _conv23_body(a_ref, dif_ref, dib_ref, y_ref, mf_ref, mb_ref, xp_ref,
                 w0_ref, w1_ref, b_ref, p_ref,
                 o_feat, o_y, o_cs, v_scr, vr_scr, xb_scr):
    i = pl.program_id(0)

    @pl.when(i == 0)
    def _():
        y = y_ref[...]
        g = mf_ref[...] / (1.0 + jnp.exp(-y))
        xg = xp_ref[...] * g
        xw = jnp.dot(xg, w0_ref[...], preferred_element_type=jnp.float32)
        vf = dif_ref[...] * xw
        v16 = vf.astype(jnp.bfloat16)
        v_scr[...] = v16
        vr_scr[...] = (vf - v16.astype(jnp.float32)).astype(jnp.bfloat16)
        xb_scr[...] = (jnp.dot(xg, w1_ref[...],
                               preferred_element_type=jnp.float32)
                       + b_ref[...])

    z = (jnp.dot(a_ref[...], v_scr[...], preferred_element_type=jnp.float32)
         + jnp.dot(a_ref[...], vr_scr[...],
                   preferred_element_type=jnp.float32))
    feat = (jax.nn.relu(dib_ref[...] * z + xb_scr[pl.ds(i * BR, BR), :])
            * mb_ref[...])
    o_feat[...] = feat
    o_y[...] = jnp.dot(feat, p_ref[...], preferred_element_type=jnp.float32)
    o_cs[...] = jnp.sum(feat, axis=0, keepdims=True)[None]


def _conv23(a16, di, y, m, xprev, w0, w1, b, p_col):
    h = xprev.shape[1]
    return pl.pallas_call(
        _conv23_body,
        grid=(N // BR,),
        in_specs=[
            pl.BlockSpec((BR, N), lambda i: (i, 0)),
            pl.BlockSpec((N, 1), lambda i: (0, 0)),
            pl.BlockSpec((BR, 1), lambda i: (i, 0)),
            pl.BlockSpec((N, 1), lambda i: (0, 0)),
            pl.BlockSpec((N, 1), lambda i: (0, 0)),
            pl.BlockSpec((BR, 1), lambda i: (i, 0)),
            pl.BlockSpec((N, h), lambda i: (0, 0)),
            pl.BlockSpec((h, h), lambda i: (0, 0)),
            pl.BlockSpec((h, h), lambda i: (0, 0)),
            pl.BlockSpec((1, h), lambda i: (0, 0)),
            pl.BlockSpec((h, 1), lambda i: (0, 0)),
        ],
        out_specs=[
            pl.BlockSpec((BR, h), lambda i: (i, 0)),
            pl.BlockSpec((BR, 1), lambda i: (i, 0)),
            pl.BlockSpec((1, 1, h), lambda i: (i, 0, 0)),
        ],
        out_shape=[
            jax.ShapeDtypeStruct((N, h), jnp.float32),
            jax.ShapeDtypeStruct((N, 1), jnp.float32),
            jax.ShapeDtypeStruct((N // BR, 1, h), jnp.float32),
        ],
        scratch_shapes=[pltpu.VMEM((N, h), jnp.bfloat16),
                        pltpu.VMEM((N, h), jnp.bfloat16),
                        pltpu.VMEM((N, h), jnp.float32)],
    )(a16, di, di, y, m, m, xprev, w0, w1, b[None, :], p_col)


def _proj_body(x_ref, w0_ref, w1_ref, b_ref, o0_ref, o1_ref):
    xb = x_ref[...]
    o0_ref[...] = jnp.dot(xb, w0_ref[...], preferred_element_type=jnp.float32)
    o1_ref[...] = (jnp.dot(xb, w1_ref[...],
                           preferred_element_type=jnp.float32) + b_ref[...])


def _proj(x, w0, w1, b):
    n, f = x.shape
    h = w0.shape[1]
    return pl.pallas_call(
        _proj_body,
        in_specs=[pl.BlockSpec((n, f), lambda: (0, 0)),
                  pl.BlockSpec((f, h), lambda: (0, 0)),
                  pl.BlockSpec((f, h), lambda: (0, 0)),
                  pl.BlockSpec((1, h), lambda: (0, 0))],
        out_specs=[pl.BlockSpec((n, h), lambda: (0, 0)),
                   pl.BlockSpec((n, h), lambda: (0, 0))],
        out_shape=[jax.ShapeDtypeStruct((n, h), jnp.float32),
                   jax.ShapeDtypeStruct((n, h), jnp.float32)],
    )(x, w0, w1, b)


NP_ROWS = 80
NP_COLS = 128


def _thresh_body(k, y_ref, valid_ref, o_ref):
    yi = jax.lax.bitcast_convert_type(y_ref[...], jnp.int32)
    key = yi ^ jnp.where(yi < 0, jnp.int32(0x7FFFFFFF), jnp.int32(0))
    u = jax.lax.bitcast_convert_type(key ^ jnp.int32(-0x80000000), jnp.uint32)
    u = jnp.where(valid_ref[...] > 0, u, jnp.uint32(0))

    def body(b, t):
        cand = t | (jnp.uint32(1) << (jnp.uint32(31) - b.astype(jnp.uint32)))
        cnt = jnp.sum((u >= cand).astype(jnp.int32))
        return jnp.where(cnt >= k, cand, t)

    t = jax.lax.fori_loop(0, 32, body, jnp.uint32(0))
    o_ref[...] = (u >= t).astype(jnp.float32)


def _topk_mask(y, k, valid):
    pad = NP_ROWS * NP_COLS - N
    y2 = jnp.pad(y, (0, pad), constant_values=-jnp.inf).reshape(NP_ROWS, NP_COLS)
    v2 = jnp.pad(valid, (0, pad)).reshape(NP_ROWS, NP_COLS)
    m = pl.pallas_call(
        functools.partial(_thresh_body, k),
        in_specs=[pl.BlockSpec((NP_ROWS, NP_COLS), lambda: (0, 0)),
                  pl.BlockSpec((NP_ROWS, NP_COLS), lambda: (0, 0))],
        out_specs=pl.BlockSpec((NP_ROWS, NP_COLS), lambda: (0, 0)),
        out_shape=jax.ShapeDtypeStruct((NP_ROWS, NP_COLS), jnp.float32),
    )(y2, v2)
    return m.reshape(-1)[:N]


def kernel(x, a, i, w0_1, w1_1, b1, p, w0_2, w1_2, b2, w0_3, w1_3, b3, wd, bd):
    del i
    pn = (p / jnp.sqrt(jnp.sum(p * p)))[:, None]

    a16, di0 = _cast_deg(a)
    xw0, xb0 = _proj(x, w0_1, w1_1, b1[None, :])
    x1f, y1, _ = _conv1(a16, di0, xw0, xb0, pn)

    k1 = int(np.ceil(0.5 * N))
    m1 = _topk_mask(y1[:, 0], k1, jnp.ones((N,), jnp.float32))
    di1 = _deg_pass(a16, m1[:, None])
    x2f, y2, _ = _conv23(a16, di1, y1, m1[:, None], x1f, w0_2, w1_2, b2, pn)

    k2 = int(np.ceil(0.5 * k1))
    m2 = _topk_mask(y2[:, 0], k2, m1)
    di2 = _deg_pass(a16, m2[:, None])
    _, _, cs = _conv23(a16, di2, y2, m2[:, None], x2f, w0_3, w1_3, b3, pn)

    pooled = jnp.sum(cs, axis=0) / k2
    return pooled @ wd + bd[None, :]


# speedup vs baseline: 1.1088x; 1.1088x over previous
import functools

import jax
import jax.numpy as jnp
import numpy as np
from jax.experimental import pallas as pl
from jax.experimental.pallas import tpu as pltpu

N = 10000
BR = 400
BRC = 400


def _dinv(deg):
    return jnp.where(deg > 0, jax.lax.rsqrt(deg), 0.0)


def _cast_deg_body(a_ref, a16_ref, di_ref):
    blk = a_ref[...]
    a16_ref[...] = blk.astype(jnp.bfloat16)
    di_ref[...] = _dinv(jnp.sum(blk, axis=1, keepdims=True))


def _cast_deg(a):
    return pl.pallas_call(
        _cast_deg_body,
        grid=(N // BRC,),
        in_specs=[pl.BlockSpec((BRC, N), lambda i: (i, 0))],
        out_specs=[pl.BlockSpec((BRC, N), lambda i: (i, 0)),
                   pl.BlockSpec((BRC, 1), lambda i: (i, 0))],
        out_shape=[jax.ShapeDtypeStruct((N, N), jnp.bfloat16),
                   jax.ShapeDtypeStruct((N, 1), jnp.float32)],
    )(a)


def _deg_body(a_ref, m_ref, o_ref):
    deg = jnp.dot(a_ref[...], m_ref[...], preferred_element_type=jnp.float32)
    o_ref[...] = _dinv(deg)


def _deg_pass(a16, m):
    return pl.pallas_call(
        _deg_body,
        grid=(N // BR,),
        in_specs=[
            pl.BlockSpec((BR, N), lambda i: (i, 0)),
            pl.BlockSpec((N, 1), lambda i: (0, 0)),
        ],
        out_specs=pl.BlockSpec((BR, 1), lambda i: (i, 0)),
        out_shape=jax.ShapeDtypeStruct((N, 1), jnp.float32),
    )(a16, m.astype(a16.dtype))


def _conv1_body(a_ref, dif_ref, dib_ref, xw_ref, xb_ref, p_ref,
                o_feat, o_y, o_cs, v_scr):
    @pl.when(pl.program_id(0) == 0)
    def _():
        v_scr[...] = (dif_ref[...] * xw_ref[...]).astype(jnp.bfloat16)

    z = jnp.dot(a_ref[...], v_scr[...], preferred_element_type=jnp.float32)
    feat = jax.nn.relu(dib_ref[...] * z + xb_ref[...])
    o_feat[...] = feat
    o_y[...] = jnp.dot(feat, p_ref[...], preferred_element_type=jnp.float32)
    o_cs[...] = jnp.sum(feat, axis=0, keepdims=True)[None]


def _conv1(a16, di, xw, xb, p_col):
    h = xw.shape[1]
    return pl.pallas_call(
        _conv1_body,
        grid=(N // BR,),
        in_specs=[
            pl.BlockSpec((BR, N), lambda i: (i, 0)),
            pl.BlockSpec((N, 1), lambda i: (0, 0)),
            pl.BlockSpec((BR, 1), lambda i: (i, 0)),
            pl.BlockSpec((N, h), lambda i: (0, 0)),
            pl.BlockSpec((BR, h), lambda i: (i, 0)),
            pl.BlockSpec((h, 1), lambda i: (0, 0)),
        ],
        out_specs=[
            pl.BlockSpec((BR, h), lambda i: (i, 0)),
            pl.BlockSpec((BR, 1), lambda i: (i, 0)),
            pl.BlockSpec((1, 1, h), lambda i: (i, 0, 0)),
        ],
        out_shape=[
            jax.ShapeDtypeStruct((N, h), jnp.float32),
            jax.ShapeDtypeStruct((N, 1), jnp.float32),
            jax.ShapeDtypeStruct((N // BR, 1, h), jnp.float32),
        ],
        scratch_shapes=[pltpu.VMEM((N, h), jnp.bfloat16)],
    )(a16, di, di, xw, xb, p_col)


def _conv23_body(a_ref, dif_ref, dib_ref, y_ref, mf_ref, mb_ref, xp_ref,
                 w0_ref, w1_ref, b_ref, p_ref,
                 o_feat, o_y, o_cs, v_scr, vr_scr, xb_scr, *, resid):
    i = pl.program_id(0)

    @pl.when(i == 0)
    def _():
        y = y_ref[...]
        g = mf_ref[...] / (1.0 + jnp.exp(-y))
        xg = xp_ref[...] * g
        xw = jnp.dot(xg, w0_ref[...], preferred_element_type=jnp.float32)
        vf = dif_ref[...] * xw
        v16 = vf.astype(jnp.bfloat16)
        v_scr[...] = v16
        vr_scr[...] = (vf - v16.astype(jnp.float32)).astype(jnp.bfloat16)
        xb_scr[...] = (jnp.dot(xg, w1_ref[...],
                               preferred_element_type=jnp.float32)
                       + b_ref[...])

    z = jnp.dot(a_ref[...], v_scr[...], preferred_element_type=jnp.float32)
    if resid:
        z = z + jnp.dot(a_ref[...], vr_scr[...],
                        preferred_element_type=jnp.float32)
    feat = (jax.nn.relu(dib_ref[...] * z + xb_scr[pl.ds(i * BR, BR), :])
            * mb_ref[...])
    o_feat[...] = feat
    o_y[...] = jnp.dot(feat, p_ref[...], preferred_element_type=jnp.float32)
    o_cs[...] = jnp.sum(feat, axis=0, keepdims=True)[None]


def _conv23(a16, di, y, m, xprev, w0, w1, b, p_col, resid=False):
    h = xprev.shape[1]
    return pl.pallas_call(
        functools.partial(_conv23_body, resid=resid),
        grid=(N // BR,),
        in_specs=[
            pl.BlockSpec((BR, N), lambda i: (i, 0)),
            pl.BlockSpec((N, 1), lambda i: (0, 0)),
            pl.BlockSpec((BR, 1), lambda i: (i, 0)),
            pl.BlockSpec((N, 1), lambda i: (0, 0)),
            pl.BlockSpec((N, 1), lambda i: (0, 0)),
            pl.BlockSpec((BR, 1), lambda i: (i, 0)),
            pl.BlockSpec((N, h), lambda i: (0, 0)),
            pl.BlockSpec((h, h), lambda i: (0, 0)),
            pl.BlockSpec((h, h), lambda i: (0, 0)),
            pl.BlockSpec((1, h), lambda i: (0, 0)),
            pl.BlockSpec((h, 1), lambda i: (0, 0)),
        ],
        out_specs=[
            pl.BlockSpec((BR, h), lambda i: (i, 0)),
            pl.BlockSpec((BR, 1), lambda i: (i, 0)),
            pl.BlockSpec((1, 1, h), lambda i: (i, 0, 0)),
        ],
        out_shape=[
            jax.ShapeDtypeStruct((N, h), jnp.float32),
            jax.ShapeDtypeStruct((N, 1), jnp.float32),
            jax.ShapeDtypeStruct((N // BR, 1, h), jnp.float32),
        ],
        scratch_shapes=[pltpu.VMEM((N, h), jnp.bfloat16),
                        pltpu.VMEM((N, h), jnp.bfloat16),
                        pltpu.VMEM((N, h), jnp.float32)],
    )(a16, di, di, y, m, m, xprev, w0, w1, b[None, :], p_col)


def _proj_body(x_ref, w0_ref, w1_ref, b_ref, o0_ref, o1_ref):
    xb = x_ref[...]
    o0_ref[...] = jnp.dot(xb, w0_ref[...], preferred_element_type=jnp.float32)
    o1_ref[...] = (jnp.dot(xb, w1_ref[...],
                           preferred_element_type=jnp.float32) + b_ref[...])


def _proj(x, w0, w1, b):
    n, f = x.shape
    h = w0.shape[1]
    return pl.pallas_call(
        _proj_body,
        in_specs=[pl.BlockSpec((n, f), lambda: (0, 0)),
                  pl.BlockSpec((f, h), lambda: (0, 0)),
                  pl.BlockSpec((f, h), lambda: (0, 0)),
                  pl.BlockSpec((1, h), lambda: (0, 0))],
        out_specs=[pl.BlockSpec((n, h), lambda: (0, 0)),
                   pl.BlockSpec((n, h), lambda: (0, 0))],
        out_shape=[jax.ShapeDtypeStruct((n, h), jnp.float32),
                   jax.ShapeDtypeStruct((n, h), jnp.float32)],
    )(x, w0, w1, b)


NP_ROWS = 80
NP_COLS = 128


def _thresh_body(k, y_ref, valid_ref, o_ref):
    yi = jax.lax.bitcast_convert_type(y_ref[...], jnp.int32)
    key = yi ^ jnp.where(yi < 0, jnp.int32(0x7FFFFFFF), jnp.int32(0))
    u = jax.lax.bitcast_convert_type(key ^ jnp.int32(-0x80000000), jnp.uint32)
    u = jnp.where(valid_ref[...] > 0, u, jnp.uint32(0))

    def body(b, t):
        cand = t | (jnp.uint32(1) << (jnp.uint32(31) - b.astype(jnp.uint32)))
        cnt = jnp.sum((u >= cand).astype(jnp.int32))
        return jnp.where(cnt >= k, cand, t)

    t = jax.lax.fori_loop(0, 32, body, jnp.uint32(0))
    o_ref[...] = (u >= t).astype(jnp.float32)


def _topk_mask(y, k, valid):
    pad = NP_ROWS * NP_COLS - N
    y2 = jnp.pad(y, (0, pad), constant_values=-jnp.inf).reshape(NP_ROWS, NP_COLS)
    v2 = jnp.pad(valid, (0, pad)).reshape(NP_ROWS, NP_COLS)
    m = pl.pallas_call(
        functools.partial(_thresh_body, k),
        in_specs=[pl.BlockSpec((NP_ROWS, NP_COLS), lambda: (0, 0)),
                  pl.BlockSpec((NP_ROWS, NP_COLS), lambda: (0, 0))],
        out_specs=pl.BlockSpec((NP_ROWS, NP_COLS), lambda: (0, 0)),
        out_shape=jax.ShapeDtypeStruct((NP_ROWS, NP_COLS), jnp.float32),
    )(y2, v2)
    return m.reshape(-1)[:N]


def kernel(x, a, i, w0_1, w1_1, b1, p, w0_2, w1_2, b2, w0_3, w1_3, b3, wd, bd):
    del i
    pn = (p / jnp.sqrt(jnp.sum(p * p)))[:, None]

    a16, di0 = _cast_deg(a)
    xw0, xb0 = _proj(x, w0_1, w1_1, b1[None, :])
    x1f, y1, _ = _conv1(a16, di0, xw0, xb0, pn)

    k1 = int(np.ceil(0.5 * N))
    m1 = _topk_mask(y1[:, 0], k1, jnp.ones((N,), jnp.float32))
    di1 = _deg_pass(a16, m1[:, None])
    x2f, y2, _ = _conv23(a16, di1, y1, m1[:, None], x1f, w0_2, w1_2, b2, pn)

    k2 = int(np.ceil(0.5 * k1))
    m2 = _topk_mask(y2[:, 0], k2, m1)
    di2 = _deg_pass(a16, m2[:, None])
    _, _, cs = _conv23(a16, di2, y2, m2[:, None], x2f, w0_3, w1_3, b3, pn,
                       resid=True)

    pooled = jnp.sum(cs, axis=0) / k2
    return pooled @ wd + bd[None, :]


# speedup vs baseline: 1.1698x; 1.0550x over previous
import functools

import jax
import jax.numpy as jnp
import numpy as np
from jax.experimental import pallas as pl
from jax.experimental.pallas import tpu as pltpu

N = 10000
BR = 400
BRC = 400


def _dinv(deg):
    return jnp.where(deg > 0, jax.lax.rsqrt(deg), 0.0)


def _cast_deg_body(a_ref, a16_ref, di_ref):
    blk = a_ref[...]
    a16_ref[...] = blk.astype(jnp.bfloat16)
    di_ref[...] = _dinv(jnp.sum(blk, axis=1, keepdims=True))


def _cast_deg(a):
    return pl.pallas_call(
        _cast_deg_body,
        grid=(N // BRC,),
        in_specs=[pl.BlockSpec((BRC, N), lambda i: (i, 0))],
        out_specs=[pl.BlockSpec((BRC, N), lambda i: (i, 0)),
                   pl.BlockSpec((BRC, 1), lambda i: (i, 0))],
        out_shape=[jax.ShapeDtypeStruct((N, N), jnp.bfloat16),
                   jax.ShapeDtypeStruct((N, 1), jnp.float32)],
    )(a)


def _deg_body(a_ref, m_ref, o_ref):
    deg = jnp.dot(a_ref[...], m_ref[...], preferred_element_type=jnp.float32)
    o_ref[...] = _dinv(deg)


def _deg_pass(a16, m):
    return pl.pallas_call(
        _deg_body,
        grid=(N // BR,),
        in_specs=[
            pl.BlockSpec((BR, N), lambda i: (i, 0)),
            pl.BlockSpec((N, 1), lambda i: (0, 0)),
        ],
        out_specs=pl.BlockSpec((BR, 1), lambda i: (i, 0)),
        out_shape=jax.ShapeDtypeStruct((N, 1), jnp.float32),
    )(a16, m.astype(a16.dtype))


def _conv1_body(a_ref, dif_ref, dib_ref, xw_ref, xb_ref, p_ref,
                o_feat, o_y, o_cs, v_scr):
    @pl.when(pl.program_id(0) == 0)
    def _():
        v_scr[...] = (dif_ref[...] * xw_ref[...]).astype(jnp.bfloat16)

    z = jnp.dot(a_ref[...], v_scr[...], preferred_element_type=jnp.float32)
    feat = jax.nn.relu(dib_ref[...] * z + xb_ref[...])
    o_feat[...] = feat
    o_y[...] = jnp.dot(feat, p_ref[...], preferred_element_type=jnp.float32)
    o_cs[...] = jnp.sum(feat, axis=0, keepdims=True)[None]


def _conv1(a16, di, xw, xb, p_col):
    h = xw.shape[1]
    return pl.pallas_call(
        _conv1_body,
        grid=(N // BR,),
        in_specs=[
            pl.BlockSpec((BR, N), lambda i: (i, 0)),
            pl.BlockSpec((N, 1), lambda i: (0, 0)),
            pl.BlockSpec((BR, 1), lambda i: (i, 0)),
            pl.BlockSpec((N, h), lambda i: (0, 0)),
            pl.BlockSpec((BR, h), lambda i: (i, 0)),
            pl.BlockSpec((h, 1), lambda i: (0, 0)),
        ],
        out_specs=[
            pl.BlockSpec((BR, h), lambda i: (i, 0)),
            pl.BlockSpec((BR, 1), lambda i: (i, 0)),
            pl.BlockSpec((1, 1, h), lambda i: (i, 0, 0)),
        ],
        out_shape=[
            jax.ShapeDtypeStruct((N, h), jnp.float32),
            jax.ShapeDtypeStruct((N, 1), jnp.float32),
            jax.ShapeDtypeStruct((N // BR, 1, h), jnp.float32),
        ],
        scratch_shapes=[pltpu.VMEM((N, h), jnp.bfloat16)],
    )(a16, di, di, xw, xb, p_col)


def _conv23_body(a_ref, dif_ref, dib_ref, y_ref, mf_ref, mb_ref, xp_ref,
                 w0_ref, w1_ref, b_ref, p_ref,
                 o_feat, o_y, o_cs, v_scr, xb_scr):
    i = pl.program_id(0)

    @pl.when(i == 0)
    def _():
        y = y_ref[...]
        g = mf_ref[...] / (1.0 + jnp.exp(-y))
        xg = xp_ref[...] * g
        xw = jnp.dot(xg, w0_ref[...], preferred_element_type=jnp.float32)
        v_scr[...] = (dif_ref[...] * xw).astype(jnp.bfloat16)
        xb_scr[...] = (jnp.dot(xg, w1_ref[...],
                               preferred_element_type=jnp.float32)
                       + b_ref[...])

    z = jnp.dot(a_ref[...], v_scr[...], preferred_element_type=jnp.float32)
    feat = (jax.nn.relu(dib_ref[...] * z + xb_scr[pl.ds(i * BR, BR), :])
            * mb_ref[...])
    o_feat[...] = feat
    o_y[...] = jnp.dot(feat, p_ref[...], preferred_element_type=jnp.float32)
    o_cs[...] = jnp.sum(feat, axis=0, keepdims=True)[None]


def _conv23(a16, di, y, m, xprev, w0, w1, b, p_col):
    h = xprev.shape[1]
    return pl.pallas_call(
        _conv23_body,
        grid=(N // BR,),
        in_specs=[
            pl.BlockSpec((BR, N), lambda i: (i, 0)),
            pl.BlockSpec((N, 1), lambda i: (0, 0)),
            pl.BlockSpec((BR, 1), lambda i: (i, 0)),
            pl.BlockSpec((N, 1), lambda i: (0, 0)),
            pl.BlockSpec((N, 1), lambda i: (0, 0)),
            pl.BlockSpec((BR, 1), lambda i: (i, 0)),
            pl.BlockSpec((N, h), lambda i: (0, 0)),
            pl.BlockSpec((h, h), lambda i: (0, 0)),
            pl.BlockSpec((h, h), lambda i: (0, 0)),
            pl.BlockSpec((1, h), lambda i: (0, 0)),
            pl.BlockSpec((h, 1), lambda i: (0, 0)),
        ],
        out_specs=[
            pl.BlockSpec((BR, h), lambda i: (i, 0)),
            pl.BlockSpec((BR, 1), lambda i: (i, 0)),
            pl.BlockSpec((1, 1, h), lambda i: (i, 0, 0)),
        ],
        out_shape=[
            jax.ShapeDtypeStruct((N, h), jnp.float32),
            jax.ShapeDtypeStruct((N, 1), jnp.float32),
            jax.ShapeDtypeStruct((N // BR, 1, h), jnp.float32),
        ],
        scratch_shapes=[pltpu.VMEM((N, h), jnp.bfloat16),
                        pltpu.VMEM((N, h), jnp.float32)],
    )(a16, di, di, y, m, m, xprev, w0, w1, b[None, :], p_col)


def _proj_body(x_ref, w0_ref, w1_ref, b_ref, o0_ref, o1_ref):
    xb = x_ref[...]
    o0_ref[...] = jnp.dot(xb, w0_ref[...], preferred_element_type=jnp.float32)
    o1_ref[...] = (jnp.dot(xb, w1_ref[...],
                           preferred_element_type=jnp.float32) + b_ref[...])


def _proj(x, w0, w1, b):
    n, f = x.shape
    h = w0.shape[1]
    return pl.pallas_call(
        _proj_body,
        in_specs=[pl.BlockSpec((n, f), lambda: (0, 0)),
                  pl.BlockSpec((f, h), lambda: (0, 0)),
                  pl.BlockSpec((f, h), lambda: (0, 0)),
                  pl.BlockSpec((1, h), lambda: (0, 0))],
        out_specs=[pl.BlockSpec((n, h), lambda: (0, 0)),
                   pl.BlockSpec((n, h), lambda: (0, 0))],
        out_shape=[jax.ShapeDtypeStruct((n, h), jnp.float32),
                   jax.ShapeDtypeStruct((n, h), jnp.float32)],
    )(x, w0, w1, b)


NP_ROWS = 80
NP_COLS = 128


def _thresh_body(k, y_ref, valid_ref, o_ref):
    yi = jax.lax.bitcast_convert_type(y_ref[...], jnp.int32)
    key = yi ^ jnp.where(yi < 0, jnp.int32(0x7FFFFFFF), jnp.int32(0))
    u = jax.lax.bitcast_convert_type(key ^ jnp.int32(-0x80000000), jnp.uint32)
    u = jnp.where(valid_ref[...] > 0, u, jnp.uint32(0))

    def body(b, t):
        cand = t | (jnp.uint32(1) << (jnp.uint32(31) - b.astype(jnp.uint32)))
        cnt = jnp.sum((u >= cand).astype(jnp.int32))
        return jnp.where(cnt >= k, cand, t)

    t = jax.lax.fori_loop(0, 32, body, jnp.uint32(0))
    o_ref[...] = (u >= t).astype(jnp.float32)


def _topk_mask(y, k, valid):
    pad = NP_ROWS * NP_COLS - N
    y2 = jnp.pad(y, (0, pad), constant_values=-jnp.inf).reshape(NP_ROWS, NP_COLS)
    v2 = jnp.pad(valid, (0, pad)).reshape(NP_ROWS, NP_COLS)
    m = pl.pallas_call(
        functools.partial(_thresh_body, k),
        in_specs=[pl.BlockSpec((NP_ROWS, NP_COLS), lambda: (0, 0)),
                  pl.BlockSpec((NP_ROWS, NP_COLS), lambda: (0, 0))],
        out_specs=pl.BlockSpec((NP_ROWS, NP_COLS), lambda: (0, 0)),
        out_shape=jax.ShapeDtypeStruct((NP_ROWS, NP_COLS), jnp.float32),
    )(y2, v2)
    return m.reshape(-1)[:N]


def kernel(x, a, i, w0_1, w1_1, b1, p, w0_2, w1_2, b2, w0_3, w1_3, b3, wd, bd):
    del i
    pn = (p / jnp.sqrt(jnp.sum(p * p)))[:, None]

    a16, di0 = _cast_deg(a)
    xw0, xb0 = _proj(x, w0_1, w1_1, b1[None, :])
    x1f, y1, _ = _conv1(a16, di0, xw0, xb0, pn)

    k1 = int(np.ceil(0.5 * N))
    m1 = _topk_mask(y1[:, 0], k1, jnp.ones((N,), jnp.float32))
    di1 = _deg_pass(a16, m1[:, None])
    x2f, y2, _ = _conv23(a16, di1, y1, m1[:, None], x1f, w0_2, w1_2, b2, pn)

    k2 = int(np.ceil(0.5 * k1))
    m2 = _topk_mask(y2[:, 0], k2, m1)
    di2 = _deg_pass(a16, m2[:, None])
    _, _, cs = _conv23(a16, di2, y2, m2[:, None], x2f, w0_3, w1_3, b3, pn)

    pooled = jnp.sum(cs, axis=0) / k2
    return pooled @ wd + bd[None, :]
